# speedup vs baseline: 1.8183x; 1.8183x over previous
import functools

import jax
import jax.numpy as jnp
from jax import lax
from jax.experimental import pallas as pl
from jax.experimental.pallas import tpu as pltpu
from jax.experimental.pallas import tpu_sc as plsc

D_MODEL = 1024
N_IDX = 4 * 8192

_info = plsc.get_sparse_core_info()
_NC, _NS = _info.num_cores, _info.num_subcores
_NW = _NC * _NS
_B_PER_W = N_IDX // _NW
_CHUNK = 32
_N_CHUNKS = _B_PER_W // _CHUNK


@functools.partial(
    pl.kernel,
    mesh=plsc.VectorSubcoreMesh(core_axis_name="c", subcore_axis_name="s"),
    out_type=jax.ShapeDtypeStruct((N_IDX, D_MODEL), jnp.float32),
    scratch_types=[
        pltpu.VMEM((_CHUNK,), jnp.int32),
        pltpu.VMEM((_CHUNK, D_MODEL), jnp.float32),
        pltpu.SemaphoreType.DMA,
    ],
)
def _gather_rows(pe_hbm, idx_hbm, out_hbm, idx_v, rows_v, sem):
    wid = lax.axis_index("s") * _NC + lax.axis_index("c")
    base = wid * _B_PER_W

    def step(g, carry):
        off = base + g * _CHUNK
        pltpu.sync_copy(idx_hbm.at[pl.ds(off, _CHUNK)], idx_v)
        pltpu.async_copy(pe_hbm.at[idx_v], rows_v, sem).wait()
        pltpu.sync_copy(rows_v, out_hbm.at[pl.ds(off, _CHUNK)])
        return carry

    lax.fori_loop(0, _N_CHUNKS, step, 0)


def kernel(t, pe):
    flat = _gather_rows(pe, t.reshape(-1).astype(jnp.int32))
    return flat.reshape(t.shape + (D_MODEL,))


# speedup vs baseline: 2.3876x; 1.3131x over previous
import functools

import jax
import jax.numpy as jnp
from jax import lax
from jax.experimental import pallas as pl
from jax.experimental.pallas import tpu as pltpu
from jax.experimental.pallas import tpu_sc as plsc

D_MODEL = 1024
N_IDX = 4 * 8192

_info = plsc.get_sparse_core_info()
_NC, _NS = _info.num_cores, _info.num_subcores
_NW = _NC * _NS
_B_PER_W = N_IDX // _NW
_CHUNK = 32
_N_CHUNKS = _B_PER_W // _CHUNK


@functools.partial(
    pl.kernel,
    mesh=plsc.VectorSubcoreMesh(core_axis_name="c", subcore_axis_name="s"),
    out_type=jax.ShapeDtypeStruct((N_IDX, D_MODEL), jnp.float32),
    scratch_types=[
        pltpu.VMEM((_B_PER_W,), jnp.int32),
        pltpu.VMEM((2, _CHUNK, D_MODEL), jnp.float32),
        pltpu.SemaphoreType.DMA,
        pltpu.SemaphoreType.DMA,
    ],
)
def _gather_rows(pe_hbm, idx_hbm, out_hbm, idx_v, rows_v, gsem, osem):
    wid = lax.axis_index("s") * _NC + lax.axis_index("c")
    base = wid * _B_PER_W

    def gather(g, buf):
        return pltpu.make_async_copy(
            pe_hbm.at[idx_v.at[pl.ds(g * _CHUNK, _CHUNK)]], rows_v.at[buf], gsem
        )

    def out_copy(g, buf):
        return pltpu.make_async_copy(
            rows_v.at[buf], out_hbm.at[pl.ds(base + g * _CHUNK, _CHUNK)], osem
        )

    pltpu.sync_copy(idx_hbm.at[pl.ds(base, _B_PER_W)], idx_v)
    gather(0, 0).start()

    def outer(i, carry):
        for b in (0, 1):
            g = 2 * i + b
            nb = 1 - b

            @pl.when(g >= 1)
            def _():
                out_copy(g - 1, nb).wait()

            @pl.when(g + 1 < _N_CHUNKS)
            def _():
                gather(g + 1, nb).start()

            gather(g, b).wait()
            out_copy(g, b).start()
        return carry

    lax.fori_loop(0, _N_CHUNKS // 2, outer, 0)
    out_copy(_N_CHUNKS - 1, 1).wait()


def kernel(t, pe):
    flat = _gather_rows(pe, t.reshape(-1).astype(jnp.int32))
    return flat.reshape(t.shape + (D_MODEL,))


# speedup vs baseline: 2.4245x; 1.0154x over previous
import functools

import jax
import jax.numpy as jnp
from jax import lax
from jax.experimental import pallas as pl
from jax.experimental.pallas import tpu as pltpu
from jax.experimental.pallas import tpu_sc as plsc

D_MODEL = 1024
N_IDX = 4 * 8192

_info = plsc.get_sparse_core_info()
_NC, _NS = _info.num_cores, _info.num_subcores
_NW = _NC * _NS
_B_PER_W = N_IDX // _NW
_CHUNK = 32
_N_CHUNKS = _B_PER_W // _CHUNK
_NBUF = 3


@functools.partial(
    pl.kernel,
    mesh=plsc.VectorSubcoreMesh(core_axis_name="c", subcore_axis_name="s"),
    out_type=jax.ShapeDtypeStruct((N_IDX, D_MODEL), jnp.float32),
    scratch_types=[
        pltpu.VMEM((_B_PER_W,), jnp.int32),
        pltpu.VMEM((_NBUF, _CHUNK, D_MODEL), jnp.float32),
        pltpu.SemaphoreType.DMA,
        pltpu.SemaphoreType.DMA,
    ],
)
def _gather_rows(pe_hbm, idx_hbm, out_hbm, idx_v, rows_v, gsem, osem):
    wid = lax.axis_index("s") * _NC + lax.axis_index("c")
    base = wid * _B_PER_W

    def gather(g, buf):
        return pltpu.make_async_copy(
            pe_hbm.at[idx_v.at[pl.ds(g * _CHUNK, _CHUNK)]], rows_v.at[buf], gsem
        )

    def out_copy(g, buf):
        return pltpu.make_async_copy(
            rows_v.at[buf], out_hbm.at[pl.ds(base + g * _CHUNK, _CHUNK)], osem
        )

    pltpu.sync_copy(idx_hbm.at[pl.ds(base, _B_PER_W)], idx_v)
    gather(0, 0).start()

    def outer(i, carry):
        for b in range(_NBUF):
            g = i * _NBUF + b

            @pl.when(g < _N_CHUNKS)
            def _():
                nb = (b + 1) % _NBUF

                @pl.when(jnp.logical_and(g + 1 < _N_CHUNKS, g + 1 >= _NBUF))
                def _():
                    out_copy(g + 1 - _NBUF, nb).wait()

                @pl.when(g + 1 < _N_CHUNKS)
                def _():
                    gather(g + 1, nb).start()

                gather(g, b).wait()
                out_copy(g, b).start()
        return carry

    lax.fori_loop(0, (_N_CHUNKS + _NBUF - 1) // _NBUF, outer, 0)
    for g in range(_N_CHUNKS - _NBUF, _N_CHUNKS):
        out_copy(g, g % _NBUF).wait()


def kernel(t, pe):
    flat = _gather_rows(pe, t.reshape(-1).astype(jnp.int32))
    return flat.reshape(t.shape + (D_MODEL,))
